# baseline (speedup 1.0000x reference)
import functools

import jax
import jax.numpy as jnp
from jax import lax
from jax.experimental import pallas as pl
from jax.experimental.pallas import tpu as pltpu, tpu_sc as plsc

N = 10000
D = 128
D_LAT = 64
G = 64

NC = 2
NS = 16
NW = NC * NS
L = 16

CHUNK = 64
NPAD = 10112
ROWS_PER_TILE = NPAD // NS
BR = 1264
def _mesh():
    return plsc.VectorSubcoreMesh(core_axis_name="c", subcore_axis_name="s",
                                  num_cores=NC, num_subcores=NS)


def _pad_edges(e):
    epw = ((e + NW * 4 * CHUNK - 1) // (NW * 4 * CHUNK)) * 4 * CHUNK
    return epw * NW, epw, epw // CHUNK


def _make_deg_kernel(epw, nchw):
    @functools.partial(
        pl.kernel,
        out_type=jax.ShapeDtypeStruct((NW, NPAD), jnp.float32),
        mesh=_mesh(),
        scratch_types=[
            pltpu.VMEM((nchw, CHUNK), jnp.int32),
            pltpu.VMEM((NPAD,), jnp.float32),
        ],
        compiler_params=pltpu.CompilerParams(needs_layout_passes=False),
    )
    def deg_kernel(dst_hbm, out_hbm, dstv, degv):
        cid = lax.axis_index("c")
        sid = lax.axis_index("s")
        wid = sid * NC + cid
        zero16 = jnp.zeros((L,), jnp.float32)

        def zbody(i, carry):
            degv[pl.ds(i * L, L)] = zero16
            return carry

        lax.fori_loop(0, NPAD // L, zbody, 0)
        pltpu.sync_copy(dst_hbm.at[wid], dstv)
        one16 = jnp.ones((L,), jnp.float32)

        def body(i, carry):
            idx = dstv[i // (CHUNK // L), pl.ds((i % (CHUNK // L)) * L, L)]
            plsc.addupdate_scatter(degv, [idx], one16)
            return carry

        lax.fori_loop(0, epw // L, body, 0)
        pltpu.sync_copy(degv, out_hbm.at[wid])

    return deg_kernel


def _make_scatter_kernel(nchw):
    @functools.partial(
        pl.kernel,
        out_type=jax.ShapeDtypeStruct((NC, NPAD, D), jnp.float32),
        mesh=_mesh(),
        scratch_types=[
            pltpu.VMEM((nchw // 2, CHUNK), jnp.int32),
            pltpu.VMEM((nchw // 2, CHUNK), jnp.int32),
            pltpu.VMEM((CHUNK, D), jnp.float32),
            pltpu.VMEM((CHUNK, D), jnp.float32),
            pltpu.VMEM_SHARED((NPAD, D), jnp.float32),
            pltpu.SemaphoreType.DMA,
            pltpu.SemaphoreType.DMA,
        ],
        compiler_params=pltpu.CompilerParams(needs_layout_passes=False),
    )
    def scatter_kernel(z_hbm, src_hbm, dst_hbm, out_hbm, srcv, dstv,
                       rows0, rows1, acc, sem0, sem1):
        cid = lax.axis_index("c")
        sid = lax.axis_index("s")
        wid = sid * NC + cid
        zero16 = jnp.zeros((L,), jnp.float32)

        def zbody(i, carry):
            rows0[i // (D // L), pl.ds((i % (D // L)) * L, L)] = zero16
            return carry

        lax.fori_loop(0, CHUNK * (D // L), zbody, 0)
        zrows = ROWS_PER_TILE // CHUNK
        ztail = ROWS_PER_TILE - zrows * CHUNK
        for k in range(zrows):
            pltpu.sync_copy(rows0, acc.at[pl.ds(sid * ROWS_PER_TILE + k * CHUNK, CHUNK)])
        pltpu.sync_copy(rows0.at[pl.ds(0, ztail)],
                        acc.at[pl.ds(sid * ROWS_PER_TILE + zrows * CHUNK, ztail)])
        plsc.subcore_barrier()

        half = nchw // 2
        for h in range(2):
            pltpu.sync_copy(src_hbm.at[wid, h], srcv)
            pltpu.sync_copy(dst_hbm.at[wid, h], dstv)
            pltpu.async_copy(z_hbm.at[srcv.at[0]], rows0, sem0)

            def body(j2, carry):
                j = j2 * 2
                pltpu.make_async_copy(z_hbm.at[srcv.at[j]], rows0, sem0).wait()
                pltpu.async_copy(z_hbm.at[srcv.at[j + 1]], rows1, sem1)
                pltpu.sync_copy(rows0, acc.at[dstv.at[j]], add=True)
                pltpu.make_async_copy(z_hbm.at[srcv.at[j + 1]], rows1, sem1).wait()

                @pl.when(j2 < half // 2 - 1)
                def _():
                    pltpu.async_copy(z_hbm.at[srcv.at[j + 2]], rows0, sem0)

                pltpu.sync_copy(rows1, acc.at[dstv.at[j + 1]], add=True)
                return carry

            lax.fori_loop(0, half // 2, body, 0)
        plsc.subcore_barrier()
        for k in range(zrows):
            sl = pl.ds(sid * ROWS_PER_TILE + k * CHUNK, CHUNK)
            pltpu.sync_copy(acc.at[sl], rows0)
            pltpu.sync_copy(rows0, out_hbm.at[cid].at[sl])
        slt = pl.ds(sid * ROWS_PER_TILE + zrows * CHUNK, ztail)
        pltpu.sync_copy(acc.at[slt], rows0.at[pl.ds(0, ztail)])
        pltpu.sync_copy(rows0.at[pl.ds(0, ztail)], out_hbm.at[cid].at[slt])

    return scatter_kernel


def _prep_body(degT_ref, x_ref, w1_ref, z1_ref, dinv_ref):
    deg = jnp.sum(degT_ref[...], axis=1, keepdims=True) + 1.0
    dinv = lax.rsqrt(deg)
    q = jnp.dot(x_ref[...], w1_ref[...], preferred_element_type=jnp.float32)
    z1_ref[...] = dinv * q
    dinv_ref[...] = jnp.broadcast_to(dinv, (BR, D))


def _mid_body(s0_ref, s1_ref, z1_ref, dinv_ref, w2_ref, b1_ref, z2_ref):
    dinv = dinv_ref[...]
    h1 = jnp.maximum(dinv * (s0_ref[...] + s1_ref[...] + z1_ref[...]) + b1_ref[...], 0.0)
    z2_ref[...] = dinv * jnp.dot(h1, w2_ref[...], preferred_element_type=jnp.float32)


def _final_body(s0_ref, s1_ref, z2_ref, dinv_ref, b2_ref, batch_ref,
                wmu_ref, bmu_ref, wlv_ref, blv_ref,
                mu_ref, lv_ref, sums, cnts):
    i = pl.program_id(0)
    h2 = jnp.maximum(
        dinv_ref[...] * (s0_ref[...] + s1_ref[...] + z2_ref[...]) + b2_ref[...], 0.0)
    gids = lax.broadcasted_iota(jnp.int32, (1, G), 1)
    oh = (batch_ref[...] == gids).astype(jnp.float32)
    part_s = lax.dot_general(oh, h2, (((0,), (0,)), ((), ())),
                             preferred_element_type=jnp.float32)
    part_c = lax.dot_general(oh, jnp.ones((BR, D), jnp.float32),
                             (((0,), (0,)), ((), ())),
                             preferred_element_type=jnp.float32)

    @pl.when(i == 0)
    def _():
        sums[...] = jnp.zeros_like(sums)
        cnts[...] = jnp.zeros_like(cnts)

    sums[...] += part_s
    cnts[...] += part_c

    @pl.when(i == pl.num_programs(0) - 1)
    def _():
        pooled = sums[...] / jnp.maximum(cnts[...], 1.0)
        mu_ref[...] = jnp.dot(pooled, wmu_ref[...],
                              preferred_element_type=jnp.float32) + bmu_ref[...]
        lv_ref[...] = jnp.dot(pooled, wlv_ref[...],
                              preferred_element_type=jnp.float32) + blv_ref[...]


def _row_spec(bs=BR):
    return pl.BlockSpec((bs, D), lambda i: (i, 0))


def _full_spec(shape):
    return pl.BlockSpec(shape, lambda i: tuple(0 for _ in shape))


def kernel(x, edge_index, batch, W1, b1, W2, b2, Wmu, bmu, Wlv, blv):
    e = edge_index.shape[1]
    e_pad, epw, nchw = _pad_edges(e)
    pad = e_pad - e

    pad_ar = lax.iota(jnp.int32, pad)
    src_p = jnp.concatenate(
        [edge_index[0], pad_ar % N]).reshape(NW, nchw, CHUNK)
    dst_p = jnp.concatenate(
        [edge_index[1], N + pad_ar % (NPAD - N)]).reshape(NW, nchw, CHUNK)
    x_p = jnp.concatenate([x, jnp.zeros((NPAD - N, D), jnp.float32)])
    batch_p = jnp.concatenate(
        [batch, jnp.full((NPAD - N,), G, jnp.int32)]).reshape(NPAD, 1)

    deg_parts = _make_deg_kernel(epw, nchw)(dst_p)
    degT = deg_parts.T

    nblk = NPAD // BR
    z1, dinv_b = pl.pallas_call(
        _prep_body,
        grid=(nblk,),
        in_specs=[
            pl.BlockSpec((BR, NW), lambda i: (i, 0)),
            _row_spec(),
            _full_spec((D, D)),
        ],
        out_specs=[_row_spec(), _row_spec()],
        out_shape=[
            jax.ShapeDtypeStruct((NPAD, D), jnp.float32),
            jax.ShapeDtypeStruct((NPAD, D), jnp.float32),
        ],
    )(degT, x_p, W1)

    src_h = src_p.reshape(NW, 2, nchw // 2, CHUNK)
    dst_h = dst_p.reshape(NW, 2, nchw // 2, CHUNK)
    scatter = _make_scatter_kernel(nchw)
    s1 = scatter(z1, src_h, dst_h)

    z2 = pl.pallas_call(
        _mid_body,
        grid=(nblk,),
        in_specs=[
            _row_spec(), _row_spec(), _row_spec(), _row_spec(),
            _full_spec((D, D)),
            _full_spec((1, D)),
        ],
        out_specs=_row_spec(),
        out_shape=jax.ShapeDtypeStruct((NPAD, D), jnp.float32),
    )(s1[0], s1[1], z1, dinv_b, W2, b1.reshape(1, D))

    s2 = scatter(z2, src_h, dst_h)

    mu, logvar = pl.pallas_call(
        _final_body,
        grid=(nblk,),
        in_specs=[
            _row_spec(), _row_spec(), _row_spec(), _row_spec(),
            _full_spec((1, D)),
            pl.BlockSpec((BR, 1), lambda i: (i, 0)),
            _full_spec((D, D_LAT)),
            _full_spec((1, D_LAT)),
            _full_spec((D, D_LAT)),
            _full_spec((1, D_LAT)),
        ],
        out_specs=[_full_spec((G, D_LAT)), _full_spec((G, D_LAT))],
        out_shape=[
            jax.ShapeDtypeStruct((G, D_LAT), jnp.float32),
            jax.ShapeDtypeStruct((G, D_LAT), jnp.float32),
        ],
        scratch_shapes=[
            pltpu.VMEM((G, D), jnp.float32),
            pltpu.VMEM((G, D), jnp.float32),
        ],
    )(s2[0], s2[1], z2, dinv_b, b2.reshape(1, D), batch_p,
      Wmu, bmu.reshape(1, D_LAT), Wlv, blv.reshape(1, D_LAT))

    return (mu, logvar)


# speedup vs baseline: 1.2620x; 1.2620x over previous
import functools

import jax
import jax.numpy as jnp
from jax import lax
from jax.experimental import pallas as pl
from jax.experimental.pallas import tpu as pltpu, tpu_sc as plsc

N = 10000
D = 128
D_LAT = 64
G = 64

NC = 2
NS = 16
NW = NC * NS
L = 16

CHUNK = 128
NPAD = 10112
ROWS_PER_TILE = NPAD // NS
BR = 1264
def _mesh():
    return plsc.VectorSubcoreMesh(core_axis_name="c", subcore_axis_name="s",
                                  num_cores=NC, num_subcores=NS)


def _pad_edges(e):
    epw = ((e + NW * 4 * CHUNK - 1) // (NW * 4 * CHUNK)) * 4 * CHUNK
    return epw * NW, epw, epw // CHUNK


def _make_deg_kernel(epw, nchw):
    @functools.partial(
        pl.kernel,
        out_type=jax.ShapeDtypeStruct((NW, NPAD), jnp.float32),
        mesh=_mesh(),
        scratch_types=[
            pltpu.VMEM((nchw, CHUNK), jnp.int32),
            pltpu.VMEM((NPAD,), jnp.float32),
        ],
        compiler_params=pltpu.CompilerParams(needs_layout_passes=False),
    )
    def deg_kernel(dst_hbm, out_hbm, dstv, degv):
        cid = lax.axis_index("c")
        sid = lax.axis_index("s")
        wid = sid * NC + cid
        zero16 = jnp.zeros((L,), jnp.float32)

        def zbody(i, carry):
            degv[pl.ds(i * L, L)] = zero16
            return carry

        lax.fori_loop(0, NPAD // L, zbody, 0)
        pltpu.sync_copy(dst_hbm.at[wid], dstv)
        one16 = jnp.ones((L,), jnp.float32)

        def body(i, carry):
            idx = dstv[i // (CHUNK // L), pl.ds((i % (CHUNK // L)) * L, L)]
            plsc.addupdate_scatter(degv, [idx], one16)
            return carry

        lax.fori_loop(0, epw // L, body, 0)
        pltpu.sync_copy(degv, out_hbm.at[wid])

    return deg_kernel


def _make_scatter_kernel(nchw):
    @functools.partial(
        pl.kernel,
        out_type=jax.ShapeDtypeStruct((NC, NPAD, D), jnp.float32),
        mesh=_mesh(),
        scratch_types=[
            pltpu.VMEM((nchw // 2, CHUNK), jnp.int32),
            pltpu.VMEM((nchw // 2, CHUNK), jnp.int32),
            pltpu.VMEM((CHUNK, D), jnp.float32),
            pltpu.VMEM((CHUNK, D), jnp.float32),
            pltpu.VMEM_SHARED((NPAD, D), jnp.float32),
            pltpu.SemaphoreType.DMA,
            pltpu.SemaphoreType.DMA,
        ],
        compiler_params=pltpu.CompilerParams(needs_layout_passes=False),
    )
    def scatter_kernel(z_hbm, src_hbm, dst_hbm, out_hbm, srcv, dstv,
                       rows0, rows1, acc, sem0, sem1):
        cid = lax.axis_index("c")
        sid = lax.axis_index("s")
        wid = sid * NC + cid
        zero16 = jnp.zeros((L,), jnp.float32)

        def zbody(i, carry):
            rows0[i // (D // L), pl.ds((i % (D // L)) * L, L)] = zero16
            return carry

        lax.fori_loop(0, CHUNK * (D // L), zbody, 0)
        zrows = ROWS_PER_TILE // CHUNK
        ztail = ROWS_PER_TILE - zrows * CHUNK
        for k in range(zrows):
            pltpu.sync_copy(rows0, acc.at[pl.ds(sid * ROWS_PER_TILE + k * CHUNK, CHUNK)])
        pltpu.sync_copy(rows0.at[pl.ds(0, ztail)],
                        acc.at[pl.ds(sid * ROWS_PER_TILE + zrows * CHUNK, ztail)])
        plsc.subcore_barrier()

        half = nchw // 2
        for h in range(2):
            pltpu.sync_copy(src_hbm.at[wid, h], srcv)
            pltpu.sync_copy(dst_hbm.at[wid, h], dstv)
            pltpu.async_copy(z_hbm.at[srcv.at[0]], rows0, sem0)

            def body(j2, carry):
                j = j2 * 2
                pltpu.make_async_copy(z_hbm.at[srcv.at[j]], rows0, sem0).wait()
                pltpu.async_copy(z_hbm.at[srcv.at[j + 1]], rows1, sem1)
                pltpu.sync_copy(rows0, acc.at[dstv.at[j]], add=True)
                pltpu.make_async_copy(z_hbm.at[srcv.at[j + 1]], rows1, sem1).wait()

                @pl.when(j2 < half // 2 - 1)
                def _():
                    pltpu.async_copy(z_hbm.at[srcv.at[j + 2]], rows0, sem0)

                pltpu.sync_copy(rows1, acc.at[dstv.at[j + 1]], add=True)
                return carry

            lax.fori_loop(0, half // 2, body, 0)
        plsc.subcore_barrier()
        for k in range(zrows):
            sl = pl.ds(sid * ROWS_PER_TILE + k * CHUNK, CHUNK)
            pltpu.sync_copy(acc.at[sl], rows0)
            pltpu.sync_copy(rows0, out_hbm.at[cid].at[sl])
        slt = pl.ds(sid * ROWS_PER_TILE + zrows * CHUNK, ztail)
        pltpu.sync_copy(acc.at[slt], rows0.at[pl.ds(0, ztail)])
        pltpu.sync_copy(rows0.at[pl.ds(0, ztail)], out_hbm.at[cid].at[slt])

    return scatter_kernel


def _prep_body(degT_ref, x_ref, w1_ref, z1_ref, dinv_ref):
    deg = jnp.sum(degT_ref[...], axis=1, keepdims=True) + 1.0
    dinv = lax.rsqrt(deg)
    q = jnp.dot(x_ref[...], w1_ref[...], preferred_element_type=jnp.float32)
    z1_ref[...] = dinv * q
    dinv_ref[...] = jnp.broadcast_to(dinv, (BR, D))


def _mid_body(s0_ref, s1_ref, z1_ref, dinv_ref, w2_ref, b1_ref, z2_ref):
    dinv = dinv_ref[...]
    h1 = jnp.maximum(dinv * (s0_ref[...] + s1_ref[...] + z1_ref[...]) + b1_ref[...], 0.0)
    z2_ref[...] = dinv * jnp.dot(h1, w2_ref[...], preferred_element_type=jnp.float32)


def _final_body(s0_ref, s1_ref, z2_ref, dinv_ref, b2_ref, batch_ref,
                wmu_ref, bmu_ref, wlv_ref, blv_ref,
                mu_ref, lv_ref, sums, cnts):
    i = pl.program_id(0)
    h2 = jnp.maximum(
        dinv_ref[...] * (s0_ref[...] + s1_ref[...] + z2_ref[...]) + b2_ref[...], 0.0)
    gids = lax.broadcasted_iota(jnp.int32, (1, G), 1)
    oh = (batch_ref[...] == gids).astype(jnp.float32)
    part_s = lax.dot_general(oh, h2, (((0,), (0,)), ((), ())),
                             preferred_element_type=jnp.float32)
    part_c = lax.dot_general(oh, jnp.ones((BR, D), jnp.float32),
                             (((0,), (0,)), ((), ())),
                             preferred_element_type=jnp.float32)

    @pl.when(i == 0)
    def _():
        sums[...] = jnp.zeros_like(sums)
        cnts[...] = jnp.zeros_like(cnts)

    sums[...] += part_s
    cnts[...] += part_c

    @pl.when(i == pl.num_programs(0) - 1)
    def _():
        pooled = sums[...] / jnp.maximum(cnts[...], 1.0)
        mu_ref[...] = jnp.dot(pooled, wmu_ref[...],
                              preferred_element_type=jnp.float32) + bmu_ref[...]
        lv_ref[...] = jnp.dot(pooled, wlv_ref[...],
                              preferred_element_type=jnp.float32) + blv_ref[...]


def _row_spec(bs=BR):
    return pl.BlockSpec((bs, D), lambda i: (i, 0))


def _full_spec(shape):
    return pl.BlockSpec(shape, lambda i: tuple(0 for _ in shape))


def kernel(x, edge_index, batch, W1, b1, W2, b2, Wmu, bmu, Wlv, blv):
    e = edge_index.shape[1]
    e_pad, epw, nchw = _pad_edges(e)
    pad = e_pad - e

    pad_ar = lax.iota(jnp.int32, pad)
    src_p = jnp.concatenate(
        [edge_index[0], pad_ar % N]).reshape(NW, nchw, CHUNK)
    dst_p = jnp.concatenate(
        [edge_index[1], N + pad_ar % (NPAD - N)]).reshape(NW, nchw, CHUNK)
    x_p = jnp.concatenate([x, jnp.zeros((NPAD - N, D), jnp.float32)])
    batch_p = jnp.concatenate(
        [batch, jnp.full((NPAD - N,), G, jnp.int32)]).reshape(NPAD, 1)

    deg_parts = _make_deg_kernel(epw, nchw)(dst_p)
    degT = deg_parts.T

    nblk = NPAD // BR
    z1, dinv_b = pl.pallas_call(
        _prep_body,
        grid=(nblk,),
        in_specs=[
            pl.BlockSpec((BR, NW), lambda i: (i, 0)),
            _row_spec(),
            _full_spec((D, D)),
        ],
        out_specs=[_row_spec(), _row_spec()],
        out_shape=[
            jax.ShapeDtypeStruct((NPAD, D), jnp.float32),
            jax.ShapeDtypeStruct((NPAD, D), jnp.float32),
        ],
    )(degT, x_p, W1)

    src_h = src_p.reshape(NW, 2, nchw // 2, CHUNK)
    dst_h = dst_p.reshape(NW, 2, nchw // 2, CHUNK)
    scatter = _make_scatter_kernel(nchw)
    s1 = scatter(z1, src_h, dst_h)

    z2 = pl.pallas_call(
        _mid_body,
        grid=(nblk,),
        in_specs=[
            _row_spec(), _row_spec(), _row_spec(), _row_spec(),
            _full_spec((D, D)),
            _full_spec((1, D)),
        ],
        out_specs=_row_spec(),
        out_shape=jax.ShapeDtypeStruct((NPAD, D), jnp.float32),
    )(s1[0], s1[1], z1, dinv_b, W2, b1.reshape(1, D))

    s2 = scatter(z2, src_h, dst_h)

    mu, logvar = pl.pallas_call(
        _final_body,
        grid=(nblk,),
        in_specs=[
            _row_spec(), _row_spec(), _row_spec(), _row_spec(),
            _full_spec((1, D)),
            pl.BlockSpec((BR, 1), lambda i: (i, 0)),
            _full_spec((D, D_LAT)),
            _full_spec((1, D_LAT)),
            _full_spec((D, D_LAT)),
            _full_spec((1, D_LAT)),
        ],
        out_specs=[_full_spec((G, D_LAT)), _full_spec((G, D_LAT))],
        out_shape=[
            jax.ShapeDtypeStruct((G, D_LAT), jnp.float32),
            jax.ShapeDtypeStruct((G, D_LAT), jnp.float32),
        ],
        scratch_shapes=[
            pltpu.VMEM((G, D), jnp.float32),
            pltpu.VMEM((G, D), jnp.float32),
        ],
    )(s2[0], s2[1], z2, dinv_b, b2.reshape(1, D), batch_p,
      Wmu, bmu.reshape(1, D_LAT), Wlv, blv.reshape(1, D_LAT))

    return (mu, logvar)
